# baseline (speedup 1.0000x reference)
import functools

import jax
import jax.numpy as jnp
import numpy as np
from jax import lax
from jax.experimental import pallas as pl
from jax.experimental.pallas import tpu as pltpu
from jax.experimental.pallas import tpu_sc as plsc

_NC = 2
_NS = 16
_L = 16
_NW = _NC * _NS

_INT_MIN = np.int32(-(2 ** 31))
_NBINS = 2048


@functools.cache
def _make_sc_kernel(B, N):
  assert B % _NW == 0 and N % _L == 0
  rpw = B // _NW
  nv = N // _L

  mesh = plsc.VectorSubcoreMesh(
      core_axis_name="c", subcore_axis_name="s",
      num_cores=_NC, num_subcores=_NS)

  @functools.partial(
      pl.kernel,
      out_type=jax.ShapeDtypeStruct((B, N), jnp.float32),
      mesh=mesh,
      scratch_types=[
          pltpu.VMEM((N,), jnp.float32),
          pltpu.VMEM((N,), jnp.float32),
          pltpu.VMEM((_NBINS,), jnp.int32),
          pltpu.VMEM((1024,), jnp.float32),
          pltpu.VMEM((rpw * _L,), jnp.int32),
          pltpu.VMEM((_L,), jnp.int32),
          pltpu.SemaphoreType.DMA,
          pltpu.SemaphoreType.DMA,
          pltpu.SemaphoreType.DMA,
          pltpu.SemaphoreType.DMA,
      ],
      compiler_params=pltpu.CompilerParams(needs_layout_passes=False),
  )
  def body(x_hbm, k_hbm, t_hbm, out_hbm, buf0, buf1, hist_v, vhist_v,
           k_v, t_v, si0, si1, so0, so1):
    wid = lax.axis_index("s") * _NC + lax.axis_index("c")
    pltpu.sync_copy(t_hbm, t_v)
    pltpu.sync_copy(k_hbm.at[wid], k_v)
    tmask = t_v[...] != 0
    zi = jnp.zeros((_L,), jnp.int32)
    zf = jnp.zeros((_L,), jnp.float32)
    ones = jnp.full((_L,), 1, jnp.int32)
    bufs = (buf0, buf1)
    insems = (si0, si1)
    outsems = (so0, so1)

    def zero_hist(i):
      hist_v[pl.ds(i * _L, _L)] = zi
    plsc.parallel_loop(0, _NBINS // _L)(zero_hist)
    def zero_vhist(i):
      vhist_v[pl.ds(i * _L, _L)] = zf
    plsc.parallel_loop(0, 1024 // _L)(zero_vhist)

    def locate(nbins, kp):
      ngroups = nbins // (16 * _L)

      def p1(i, accs):
        return tuple(
            accs[g] + hist_v[pl.ds((g * 16 + i) * _L, _L)]
            for g in range(ngroups))
      accs = plsc.parallel_loop(
          0, 16, unroll=2, carry=(zi,) * ngroups)(p1)
      gsums = [jnp.broadcast_to(jnp.sum(a), (_L,)) for a in accs]

      off = zi
      jg = zi
      roff = zi
      for g in range(ngroups):
        off = off + gsums[g]
        below = off <= kp
        jg = jg + jnp.where(below, np.int32(1), np.int32(0))
        roff = roff + jnp.where(below, gsums[g], zi)
      base = lax.shift_right_logical(jnp.sum(jg), np.int32(4)) * 256

      def p2(i, carry):
        run, bacc, cbacc = carry
        h = hist_v[pl.ds(base + i * _L, _L)]
        s = run + plsc.cumsum(h)
        m = s <= kp
        bacc = bacc + plsc.all_reduce_population_count(m)
        cbacc = cbacc + jnp.where(m, h, np.int32(0))
        run = run + jnp.broadcast_to(jnp.sum(h), (_L,))
        return run, bacc, cbacc
      _, bacc, cbacc = plsc.parallel_loop(0, 16, carry=(roff, zi, zi))(p2)

      b = jg * np.int32(256) + bacc
      cbelow = roff + jnp.broadcast_to(jnp.sum(cbacc), (_L,))

      def pz(i):
        hist_v[pl.ds(i * _L, _L)] = zi
      plsc.parallel_loop(0, nbins // _L, unroll=8)(pz)
      return b, kp - cbelow

    def locate3(kp):
      b, _ = locate(1024, kp)

      def pt(i, acc):
        vs = vhist_v[pl.ds(i * _L, _L)]
        vhist_v[pl.ds(i * _L, _L)] = zf
        idx = lax.iota(jnp.int32, 16) + i * np.int32(_L)
        return acc + jnp.where(idx >= b, vs, np.float32(0.0))
      dacc = plsc.parallel_loop(0, 1024 // _L, unroll=4, carry=zf)(pt)
      return b, dacc

    out_descs = [None] * rpw
    pltpu.async_copy(x_hbm.at[wid * rpw], bufs[0], insems[0])

    for r in range(rpw):
      buf = bufs[r % 2]
      nxt = (r + 1) % 2
      if r + 1 < rpw:
        if r >= 1:
          out_descs[r - 1].wait()
        pltpu.async_copy(x_hbm.at[wid * rpw + r + 1], bufs[nxt],
                         insems[nxt])
      pltpu.make_async_copy(x_hbm.at[wid * rpw + r], buf,
                            insems[r % 2]).wait()
      kp = k_v[pl.ds(r * _L, _L)]

      def pass_a(i, acc, buf=buf):
        v = buf[pl.ds(i * _L, _L)]
        b = plsc.bitcast(v, jnp.int32)
        ub = jnp.where(b >= 0, jnp.bitwise_xor(b, _INT_MIN),
                       jnp.bitwise_not(b))
        buf[pl.ds(i * _L, _L)] = plsc.bitcast(ub, jnp.float32)
        d = lax.shift_right_logical(ub, np.int32(21))
        plsc.addupdate_scatter(hist_v, [d], ones)
        return acc + v
      na = plsc.parallel_loop(0, nv, unroll=8, carry=zf)(pass_a)
      num = jnp.broadcast_to(jnp.sum(na), (_L,))

      b1, k2 = locate(2048, kp)

      def pass_h2(i, buf=buf):
        ub = plsc.bitcast(buf[pl.ds(i * _L, _L)], jnp.int32)
        m = lax.shift_right_logical(ub, np.int32(21)) == b1
        d = jnp.bitwise_and(lax.shift_right_logical(ub, np.int32(10)),
                            np.int32(0x7FF))
        plsc.addupdate_scatter(hist_v, [d], ones, mask=m)
      plsc.parallel_loop(0, nv, unroll=8)(pass_h2)

      b2, k3 = locate(2048, k2)
      hi21 = jnp.bitwise_or(lax.shift_left(b1, np.int32(11)), b2)

      def pass_h3(i, acc, buf=buf):
        ub = plsc.bitcast(buf[pl.ds(i * _L, _L)], jnp.int32)
        v = plsc.bitcast(
            jnp.where(ub >= 0, jnp.bitwise_not(ub),
                      jnp.bitwise_xor(ub, _INT_MIN)), jnp.float32)
        hi22 = lax.shift_right_logical(ub, np.int32(10))
        m_eq = hi22 == hi21
        d = jnp.bitwise_and(ub, np.int32(0x3FF))
        plsc.addupdate_scatter(hist_v, [d], ones, mask=m_eq)
        plsc.addupdate_scatter(vhist_v, [d], v, mask=m_eq)
        return acc + jnp.where(hi22 > hi21, v, np.float32(0.0))
      gt = plsc.parallel_loop(0, nv, unroll=8, carry=zf)(pass_h3)

      b3, dacc = locate3(k3)
      den = jnp.broadcast_to(jnp.sum(dacc + gt), (_L,))

      thresh = jnp.bitwise_xor(
          jnp.bitwise_or(lax.shift_left(hi21, np.int32(10)), b3), _INT_MIN)

      scale = jnp.abs(jnp.where(den == 0.0, np.float32(0.0), num / den))

      def pass_d(i, buf=buf):
        ub = plsc.bitcast(buf[pl.ds(i * _L, _L)], jnp.int32)
        v = plsc.bitcast(
            jnp.where(ub >= 0, jnp.bitwise_not(ub),
                      jnp.bitwise_xor(ub, _INT_MIN)), jnp.float32)
        sk = jnp.bitwise_xor(ub, _INT_MIN)
        masked = jnp.where(sk >= thresh, v, np.float32(0.0))
        buf[pl.ds(i * _L, _L)] = jnp.where(tmask, scale * masked, v)
      plsc.parallel_loop(0, nv, unroll=8)(pass_d)

      out_descs[r] = pltpu.async_copy(buf, out_hbm.at[wid * rpw + r],
                                      outsems[r % 2])

    if rpw >= 2:
      out_descs[rpw - 2].wait()
    out_descs[rpw - 1].wait()

  return body


def kernel(inputs, training):
  B, N = inputs.shape
  probs = jax.random.uniform(
      jax.random.fold_in(jax.random.key(0), 1), (B,),
      minval=0.0, maxval=1.0)
  k = jnp.maximum(
      jnp.ceil(np.float32(N) * probs).astype(jnp.int32) - 1, 0)
  rpw = B // _NW
  k16 = jnp.broadcast_to(k[:, None], (B, _L)).astype(jnp.int32)
  karr = k16.reshape(_NW, rpw * _L)
  t16 = jnp.full((_L,), training, dtype=jnp.int32)
  return _make_sc_kernel(B, N)(inputs, karr, t16)


# speedup vs baseline: 1.0223x; 1.0223x over previous
import functools

import jax
import jax.numpy as jnp
import numpy as np
from jax import lax
from jax.experimental import pallas as pl
from jax.experimental.pallas import tpu as pltpu
from jax.experimental.pallas import tpu_sc as plsc

_NC = 2
_NS = 16
_L = 16
_NW = _NC * _NS

_INT_MIN = np.int32(-(2 ** 31))
_NBINS = 2048


@functools.cache
def _make_sc_kernel(B, N):
  assert B % _NW == 0 and N % _L == 0
  rpw = B // _NW
  nv = N // _L

  mesh = plsc.VectorSubcoreMesh(
      core_axis_name="c", subcore_axis_name="s",
      num_cores=_NC, num_subcores=_NS)

  @functools.partial(
      pl.kernel,
      out_type=jax.ShapeDtypeStruct((B, N), jnp.float32),
      mesh=mesh,
      scratch_types=[
          pltpu.VMEM((N,), jnp.float32),
          pltpu.VMEM((N,), jnp.float32),
          pltpu.VMEM((_NBINS,), jnp.int32),
          pltpu.VMEM((rpw * _L,), jnp.int32),
          pltpu.VMEM((_L,), jnp.int32),
          pltpu.SemaphoreType.DMA,
          pltpu.SemaphoreType.DMA,
          pltpu.SemaphoreType.DMA,
          pltpu.SemaphoreType.DMA,
      ],
      compiler_params=pltpu.CompilerParams(needs_layout_passes=False),
  )
  def body(x_hbm, k_hbm, t_hbm, out_hbm, buf0, buf1, hist_v,
           k_v, t_v, si0, si1, so0, so1):
    wid = lax.axis_index("s") * _NC + lax.axis_index("c")
    pltpu.sync_copy(t_hbm, t_v)
    pltpu.sync_copy(k_hbm.at[wid], k_v)
    tmask = t_v[...] != 0
    zi = jnp.zeros((_L,), jnp.int32)
    zf = jnp.zeros((_L,), jnp.float32)
    ones = jnp.full((_L,), 1, jnp.int32)
    bufs = (buf0, buf1)
    insems = (si0, si1)
    outsems = (so0, so1)

    def zero_hist(i):
      hist_v[pl.ds(i * _L, _L)] = zi
    plsc.parallel_loop(0, _NBINS // _L)(zero_hist)

    def locate(nbins, kp):
      def lbody(i, carry):
        run, bacc, cbacc = carry
        h = hist_v[pl.ds(i * _L, _L)]
        hist_v[pl.ds(i * _L, _L)] = zi
        s = run + plsc.cumsum(h)
        m = s <= kp
        bacc = bacc + plsc.all_reduce_population_count(m)
        cbacc = cbacc + jnp.where(m, h, np.int32(0))
        run = run + jnp.broadcast_to(jnp.sum(h), (_L,))
        return run, bacc, cbacc
      _, b, cbacc = plsc.parallel_loop(
          0, nbins // _L, carry=(zi, zi, zi))(lbody)
      cbelow = jnp.broadcast_to(jnp.sum(cbacc), (_L,))
      return b, kp - cbelow

    def locate3(kp, hi21):
      hi_bits = lax.shift_left(hi21, np.int32(10))
      def lbody(i, carry):
        run, bacc, dacc = carry
        h = hist_v[pl.ds(i * _L, _L)]
        hist_v[pl.ds(i * _L, _L)] = zi
        s = run + plsc.cumsum(h)
        m = s <= kp
        bacc = bacc + plsc.all_reduce_population_count(m)
        idx = lax.iota(jnp.int32, 16) + i * np.int32(_L)
        keyb = jnp.bitwise_or(hi_bits, idx)
        bv = plsc.bitcast(
            jnp.where(keyb >= 0, jnp.bitwise_not(keyb),
                      jnp.bitwise_xor(keyb, _INT_MIN)), jnp.float32)
        dacc = dacc + jnp.where(m, np.float32(0.0),
                                h.astype(jnp.float32) * bv)
        run = run + jnp.broadcast_to(jnp.sum(h), (_L,))
        return run, bacc, dacc
      _, b, dacc = plsc.parallel_loop(
          0, 1024 // _L, carry=(zi, zi, zf))(lbody)
      return b, dacc

    out_descs = [None] * rpw
    pltpu.async_copy(x_hbm.at[wid * rpw], bufs[0], insems[0])

    for r in range(rpw):
      buf = bufs[r % 2]
      nxt = (r + 1) % 2
      if r + 1 < rpw:
        if r >= 1:
          out_descs[r - 1].wait()
        pltpu.async_copy(x_hbm.at[wid * rpw + r + 1], bufs[nxt],
                         insems[nxt])
      pltpu.make_async_copy(x_hbm.at[wid * rpw + r], buf,
                            insems[r % 2]).wait()
      kp = k_v[pl.ds(r * _L, _L)]

      def pass_a(i, acc, buf=buf):
        v = buf[pl.ds(i * _L, _L)]
        b = plsc.bitcast(v, jnp.int32)
        ub = jnp.where(b >= 0, jnp.bitwise_xor(b, _INT_MIN),
                       jnp.bitwise_not(b))
        buf[pl.ds(i * _L, _L)] = plsc.bitcast(ub, jnp.float32)
        d = lax.shift_right_logical(ub, np.int32(21))
        cnts, last = plsc.scan_count(d)
        plsc.addupdate_scatter(hist_v, [d], cnts, mask=last)
        return acc + v
      na = plsc.parallel_loop(0, nv, unroll=8, carry=zf)(pass_a)
      num = jnp.broadcast_to(jnp.sum(na), (_L,))

      b1, k2 = locate(2048, kp)

      def pass_h2(i, buf=buf):
        ub = plsc.bitcast(buf[pl.ds(i * _L, _L)], jnp.int32)
        m = lax.shift_right_logical(ub, np.int32(21)) == b1
        d = jnp.bitwise_and(lax.shift_right_logical(ub, np.int32(10)),
                            np.int32(0x7FF))
        plsc.addupdate_scatter(hist_v, [d], ones, mask=m)
      plsc.parallel_loop(0, nv, unroll=8)(pass_h2)

      b2, k3 = locate(2048, k2)
      hi21 = jnp.bitwise_or(lax.shift_left(b1, np.int32(11)), b2)

      def pass_h3(i, acc, buf=buf):
        ub = plsc.bitcast(buf[pl.ds(i * _L, _L)], jnp.int32)
        v = plsc.bitcast(
            jnp.where(ub >= 0, jnp.bitwise_not(ub),
                      jnp.bitwise_xor(ub, _INT_MIN)), jnp.float32)
        hi22 = lax.shift_right_logical(ub, np.int32(10))
        m_eq = hi22 == hi21
        d = jnp.bitwise_and(ub, np.int32(0x3FF))
        plsc.addupdate_scatter(hist_v, [d], ones, mask=m_eq)
        return acc + jnp.where(hi22 > hi21, v, np.float32(0.0))
      gt = plsc.parallel_loop(0, nv, unroll=8, carry=zf)(pass_h3)

      b3, dacc = locate3(k3, hi21)
      den = jnp.broadcast_to(jnp.sum(dacc + gt), (_L,))

      thresh = jnp.bitwise_xor(
          jnp.bitwise_or(lax.shift_left(hi21, np.int32(10)), b3), _INT_MIN)

      scale = jnp.abs(jnp.where(den == 0.0, np.float32(0.0), num / den))

      def pass_d(i, buf=buf):
        ub = plsc.bitcast(buf[pl.ds(i * _L, _L)], jnp.int32)
        v = plsc.bitcast(
            jnp.where(ub >= 0, jnp.bitwise_not(ub),
                      jnp.bitwise_xor(ub, _INT_MIN)), jnp.float32)
        sk = jnp.bitwise_xor(ub, _INT_MIN)
        masked = jnp.where(sk >= thresh, v, np.float32(0.0))
        buf[pl.ds(i * _L, _L)] = jnp.where(tmask, scale * masked, v)
      plsc.parallel_loop(0, nv, unroll=8)(pass_d)

      out_descs[r] = pltpu.async_copy(buf, out_hbm.at[wid * rpw + r],
                                      outsems[r % 2])

    if rpw >= 2:
      out_descs[rpw - 2].wait()
    out_descs[rpw - 1].wait()

  return body


def kernel(inputs, training):
  B, N = inputs.shape
  probs = jax.random.uniform(
      jax.random.fold_in(jax.random.key(0), 1), (B,),
      minval=0.0, maxval=1.0)
  k = jnp.maximum(
      jnp.ceil(np.float32(N) * probs).astype(jnp.int32) - 1, 0)
  rpw = B // _NW
  k16 = jnp.broadcast_to(k[:, None], (B, _L)).astype(jnp.int32)
  karr = k16.reshape(_NW, rpw * _L)
  t16 = jnp.full((_L,), training, dtype=jnp.int32)
  return _make_sc_kernel(B, N)(inputs, karr, t16)
